# baseline (speedup 1.0000x reference)
import functools

import jax
import jax.numpy as jnp
from jax import lax
from jax.experimental import pallas as pl
from jax.experimental.pallas import tpu as pltpu
from jax.experimental.pallas import tpu_sc as plsc

N = 10000
NP = 10112
E = 320000
EP = 327680
CH = 128
NCHUNK = EP // CH
D = 128
NC = 2
NS = 16
RPT = NP // NS
W0 = 96
W1 = (NCHUNK - NS * W0) // NS
PH = 2
PC0 = W0 // PH
PC1 = W1 // PH
PCM = max(PC0, PC1)
T_CHUNKS = NCHUNK // NS
RB = NP // 4
G = NP // RB

_mesh = plsc.VectorSubcoreMesh(
    core_axis_name="c", subcore_axis_name="s", num_cores=NC, num_subcores=NS)


@functools.partial(
    pl.kernel,
    out_type=jax.ShapeDtypeStruct((NC * NP, D), jnp.float32),
    mesh=_mesh,
    scratch_types=[
        pltpu.VMEM_SHARED((NP, D), jnp.float32),
        pltpu.VMEM((T_CHUNKS, CH), jnp.int32),
        pltpu.VMEM((CH, D), jnp.float32),
        pltpu.SemaphoreType.DMA,
    ],
)
def _deg_kernel(edges_hbm, out_hbm, acc, idx_v, ones_v, sem):
    c = lax.axis_index("c")
    s = lax.axis_index("s")

    def fill(val):
        def body(i, carry):
            for k in range(D // 16):
                ones_v[i, pl.ds(k * 16, 16)] = jnp.full((16,), val, jnp.float32)
            return carry
        lax.fori_loop(0, CH, body, 0)

    base = s * RPT
    fill(0.0)
    for k in range(RPT // CH):
        pltpu.sync_copy(ones_v, acc.at[pl.ds(base + k * CH, CH)])
    rem = RPT - (RPT // CH) * CH
    pltpu.sync_copy(ones_v.at[pl.ds(0, rem)],
                    acc.at[pl.ds(base + (RPT // CH) * CH, rem)])
    fill(1.0)
    start = c * NCHUNK + s * T_CHUNKS
    pltpu.sync_copy(edges_hbm.at[pl.ds(start, T_CHUNKS)], idx_v)
    plsc.subcore_barrier()

    depth = 8

    def fire(j):
        pltpu.async_copy(ones_v, acc.at[idx_v.at[j]], sem, add=True)

    def drain():
        pltpu.make_async_copy(ones_v, acc.at[idx_v.at[0]], sem).wait()

    for j in range(depth):
        fire(j)

    def body(j, carry):
        drain()
        fire(j + depth)
        return carry

    lax.fori_loop(0, T_CHUNKS - depth, body, 0)
    for _ in range(depth):
        drain()
    plsc.subcore_barrier()
    pltpu.sync_copy(acc.at[pl.ds(base, RPT)],
                    out_hbm.at[pl.ds(c * NP + base, RPT)])


@functools.partial(
    pl.kernel,
    out_type=jax.ShapeDtypeStruct((NC, NP, D), jnp.float32),
    mesh=_mesh,
    scratch_types=[
        pltpu.VMEM_SHARED((NP, D), jnp.float32),
        pltpu.VMEM((PCM, CH), jnp.int32),
        pltpu.VMEM((PCM, CH), jnp.int32),
        pltpu.VMEM((CH, D), jnp.float32),
        pltpu.VMEM((CH, D), jnp.float32),
        pltpu.SemaphoreType.DMA,
        pltpu.SemaphoreType.DMA,
        pltpu.SemaphoreType.DMA,
        pltpu.SemaphoreType.DMA,
    ],
)
def _spmm_kernel(y_hbm, srcs_hbm, dsts_hbm, out_hbm, acc, idx_s, idx_d,
                 r0, r1, sg0, sg1, ss0, ss1):
    c = lax.axis_index("c")
    s = lax.axis_index("s")
    rows = (r0, r1)
    semg = (sg0, sg1)
    sems = (ss0, ss1)

    def fill_zero(i, carry):
        for k in range(D // 16):
            r0[i, pl.ds(k * 16, 16)] = jnp.zeros((16,), jnp.float32)
        return carry

    lax.fori_loop(0, CH, fill_zero, 0)
    base = s * RPT
    for k in range(RPT // CH):
        pltpu.sync_copy(r0, acc.at[pl.ds(base + k * CH, CH)])
    rem = RPT - (RPT // CH) * CH
    pltpu.sync_copy(r0.at[pl.ds(0, rem)],
                    acc.at[pl.ds(base + (RPT // CH) * CH, rem)])

    plsc.subcore_barrier()

    def sgather(j, b, sem):
        pltpu.async_copy(y_hbm.at[idx_s.at[j]], rows[b], sem)

    def wgather(b, sem):
        pltpu.make_async_copy(y_hbm.at[idx_s.at[0]], rows[b], sem).wait()

    def sscatter(j, b, sem):
        pltpu.async_copy(rows[b], acc.at[idx_d.at[j]], sem, add=True)

    def wscatter(b, sem):
        pltpu.make_async_copy(rows[b], acc.at[idx_d.at[0]], sem).wait()

    def run(pc, start):
        for ph in range(PH):
            pltpu.sync_copy(srcs_hbm.at[pl.ds(start + ph * pc, pc)],
                            idx_s.at[pl.ds(0, pc)])
            pltpu.sync_copy(dsts_hbm.at[pl.ds(start + ph * pc, pc)],
                            idx_d.at[pl.ds(0, pc)])
            sgather(0, 0, sg0)
            sgather(1, 1, sg1)

            def body(t, carry):
                for p in range(2):
                    j = 2 * t + p
                    wgather(p, semg[p])
                    sscatter(j, p, sems[p])
                    wscatter(p, sems[p])
                    sgather(jnp.minimum(j + 2, pc - 1), p, semg[p])
                return carry

            lax.fori_loop(0, pc // 2, body, 0)
            wgather(0, sg0)
            wgather(1, sg1)

    @pl.when(c == 0)
    def _core0():
        run(PC0, s * W0)

    @pl.when(c == 1)
    def _core1():
        run(PC1, NS * W0 + s * W1)

    plsc.subcore_barrier()
    pltpu.sync_copy(acc.at[pl.ds(base, RPT)], out_hbm.at[c, pl.ds(base, RPT)])


def _norms(deg_blk):
    d_src = jnp.sum(deg_blk[0], axis=1) * (1.0 / D)
    d_dst = jnp.sum(deg_blk[1], axis=1) * (1.0 / D)
    n_src = lax.rsqrt(jnp.maximum(d_src, 1.0))
    n_dst = lax.rsqrt(jnp.maximum(d_dst, 1.0))
    return n_src, n_dst


def _tc0_body(f_ref, w_ref, o_ref):
    o_ref[...] = jnp.dot(f_ref[...], w_ref[...],
                         preferred_element_type=jnp.float32)


def _tcs_body(y_ref, deg_ref, o_ref):
    n_src, _ = _norms(deg_ref[...])
    o_ref[...] = n_src[:, None] * y_ref[...]


def _tc2_body(r_ref, deg_ref, b_ref, w_ref, o_ref):
    i = pl.program_id(0)
    n_src, n_dst = _norms(deg_ref[...])
    h = jnp.maximum(n_dst[:, None] * (r_ref[0] + r_ref[1]) + b_ref[...], 0.0)
    rowid = lax.broadcasted_iota(jnp.int32, (RB, 1), 0) + i * RB
    h = jnp.where(rowid < N, h, 0.0)
    y = jnp.dot(h, w_ref[...], preferred_element_type=jnp.float32)
    o_ref[...] = n_src[:, None] * y


def _tc3_body(r_ref, deg_ref, b_ref, wf1_ref, bf1_ref, wf2_ref, bf2_ref,
              o_ref, acc_ref):
    i = pl.program_id(0)
    _, n_dst = _norms(deg_ref[...])
    h = jnp.maximum(n_dst[:, None] * (r_ref[0] + r_ref[1]) + b_ref[...], 0.0)
    rowid = lax.broadcasted_iota(jnp.int32, (RB, 1), 0) + i * RB
    h = jnp.where(rowid < N, h, 0.0)
    part = jnp.sum(h, axis=0, keepdims=True)

    @pl.when(i == 0)
    def _init():
        acc_ref[...] = part

    @pl.when(i > 0)
    def _accum():
        acc_ref[...] = acc_ref[...] + part

    @pl.when(i == G - 1)
    def _head():
        g = acc_ref[...] * (1.0 / N)
        x = jnp.dot(g, wf1_ref[...], preferred_element_type=jnp.float32)
        x = jnp.maximum(x + bf1_ref[...], 0.0)
        z = jnp.dot(x, wf2_ref[...], preferred_element_type=jnp.float32)
        z = z + bf2_ref[...]
        lane = lax.broadcasted_iota(jnp.int32, (1, D), 1)
        zm = jnp.where(lane < 10, z, -1e30)
        zmax = jnp.max(zm, axis=1, keepdims=True)
        ez = jnp.exp(zm - zmax)
        o_ref[...] = ez / jnp.sum(ez, axis=1, keepdims=True)


def _tc0(feats_p, W1):
    return pl.pallas_call(
        _tc0_body,
        grid=(G,),
        in_specs=[
            pl.BlockSpec((RB, D), lambda i: (i, 0)),
            pl.BlockSpec((D, D), lambda i: (0, 0)),
        ],
        out_specs=pl.BlockSpec((RB, D), lambda i: (i, 0)),
        out_shape=jax.ShapeDtypeStruct((NP, D), jnp.float32),
    )(feats_p, W1)


def _tcs(yraw, degp):
    return pl.pallas_call(
        _tcs_body,
        grid=(G,),
        in_specs=[
            pl.BlockSpec((RB, D), lambda i: (i, 0)),
            pl.BlockSpec((2, RB, D), lambda i: (0, i, 0)),
        ],
        out_specs=pl.BlockSpec((RB, D), lambda i: (i, 0)),
        out_shape=jax.ShapeDtypeStruct((NP, D), jnp.float32),
    )(yraw, degp)


def _tc2(r1, degp, b1, W2):
    return pl.pallas_call(
        _tc2_body,
        grid=(G,),
        in_specs=[
            pl.BlockSpec((2, RB, D), lambda i: (0, i, 0)),
            pl.BlockSpec((2, RB, D), lambda i: (0, i, 0)),
            pl.BlockSpec((1, D), lambda i: (0, 0)),
            pl.BlockSpec((D, D), lambda i: (0, 0)),
        ],
        out_specs=pl.BlockSpec((RB, D), lambda i: (i, 0)),
        out_shape=jax.ShapeDtypeStruct((NP, D), jnp.float32),
    )(r1, degp, b1, W2)


def _tc3(r2, degp, b2, Wf1p, bf1p, Wf2p, bf2p):
    return pl.pallas_call(
        _tc3_body,
        grid=(G,),
        in_specs=[
            pl.BlockSpec((2, RB, D), lambda i: (0, i, 0)),
            pl.BlockSpec((2, RB, D), lambda i: (0, i, 0)),
            pl.BlockSpec((1, D), lambda i: (0, 0)),
            pl.BlockSpec((D, D), lambda i: (0, 0)),
            pl.BlockSpec((1, D), lambda i: (0, 0)),
            pl.BlockSpec((D, D), lambda i: (0, 0)),
            pl.BlockSpec((1, D), lambda i: (0, 0)),
        ],
        out_specs=pl.BlockSpec((1, D), lambda i: (0, 0)),
        out_shape=jax.ShapeDtypeStruct((1, D), jnp.float32),
        scratch_shapes=[pltpu.VMEM((1, D), jnp.float32)],
    )(r2, degp, b2, Wf1p, bf1p, Wf2p, bf2p)


def kernel(feats, edge_index, W1, b1, W2, b2, Wf1, bf1, Wf2, bf2):
    src = edge_index[0]
    dst = edge_index[1]
    pad = jnp.full((EP - E,), N, dtype=jnp.int32)
    srcs = jnp.concatenate([src, pad]).reshape(NCHUNK, CH)
    dsts = jnp.concatenate([dst, pad]).reshape(NCHUNK, CH)
    edges2d = jnp.concatenate([srcs, dsts], axis=0)
    degp = _deg_kernel(edges2d).reshape(NC, NP, D)

    feats_p = jnp.pad(feats, ((0, NP - N), (0, 0)))
    yraw = _tc0(feats_p, W1)
    y1 = _tcs(yraw, degp)
    r1 = _spmm_kernel(y1, srcs, dsts)
    y2 = _tc2(r1, degp, b1.reshape(1, D), W2)
    r2 = _spmm_kernel(y2, srcs, dsts)

    Wf1p = jnp.zeros((D, D), jnp.float32).at[:, :10].set(Wf1)
    bf1p = jnp.zeros((1, D), jnp.float32).at[0, :10].set(bf1)
    Wf2p = jnp.zeros((D, D), jnp.float32).at[:10, :10].set(Wf2)
    bf2p = jnp.zeros((1, D), jnp.float32).at[0, :10].set(bf2)
    out = _tc3(r2, degp, b2.reshape(1, D), Wf1p, bf1p, Wf2p, bf2p)
    return out[:, :10]
